# baseline (speedup 1.0000x reference)
import functools

import jax
import jax.numpy as jnp
from jax import lax
from jax.experimental import pallas as pl
from jax.experimental.pallas import tpu as pltpu
from jax.experimental.pallas import tpu_sc as plsc

_VOCAB = 1000000
_DIM = 64
_B = 16384
_P = 5
_N = 20
_NPAIR = _P + _N
_ROWS = 32

_NC = 2
_NS = 16
_NW = _NC * _NS
_BPW = _B // _NW
_C = 64
_NITER = _BPW // _C

_TV = 2048
_TGRID = 245
_H = _TV * _TGRID


def _tr_kernel(a1_ref, a2_ref, b1_ref, b2_ref, ao_ref, bo_ref):
    eye = jnp.eye(_DIM, dtype=jnp.float32)

    def tr(x):
        return lax.dot_general(
            x, eye, (((0,), (0,)), ((), ())),
            preferred_element_type=jnp.float32)

    ao_ref[...] = jnp.concatenate([tr(a1_ref[...]), tr(a2_ref[...])], axis=1)
    bo_ref[...] = jnp.concatenate([tr(b1_ref[...]), tr(b2_ref[...])], axis=1)


def _transpose_tables(in_t, out_t):
    nblk = pl.cdiv(_VOCAB, _TV)
    lo_spec = pl.BlockSpec((_DIM, _TV), lambda i: (0, i))
    hi_spec = pl.BlockSpec(
        (_DIM, _TV), lambda i: (0, jnp.minimum(_TGRID + i, nblk - 1)))
    out_spec = pl.BlockSpec((_TV, 2 * _DIM), lambda i: (i, 0))
    return pl.pallas_call(
        _tr_kernel,
        grid=(_TGRID,),
        in_specs=[lo_spec, hi_spec, lo_spec, hi_spec],
        out_specs=[out_spec, out_spec],
        out_shape=[
            jax.ShapeDtypeStruct((_H, 2 * _DIM), jnp.float32),
            jax.ShapeDtypeStruct((_H, 2 * _DIM), jnp.float32),
        ],
    )(in_t, in_t, out_t, out_t)


def _remap(buf, n):
    def body(i, _):
        v = buf[pl.ds(i * 16, 16)]
        r = v + v
        buf[pl.ds(i * 16, 16)] = jnp.where(v < _H, r, r - (2 * _H - 1))
        return _
    lax.fori_loop(0, n // 16, body, 0)


_LANE15 = 15


def _sc_logits_kernel(center_hbm, pos_hbm, neg_hbm, in_embed, out_embed,
                      sl_out, cidx, pidx, nidx, crows, prows, nrows, lg, sem):
    wid = lax.axis_index("s") * _NC + lax.axis_index("c")

    zero16 = jnp.zeros((16,), jnp.float32)
    for r in range(_NPAIR, _ROWS):
        for c in range(2 * _C // 16):
            lg[r, pl.ds(c * 16, 16)] = zero16

    lane = lax.iota(jnp.int32, 16)
    last = lane == _LANE15

    def body(it, _):
        for h in range(2):
            t = it * 2 + h
            gbase = pl.multiple_of(wid * _BPW + t * _C, _C)

            pltpu.sync_copy(center_hbm.at[pl.ds(gbase, _C)], cidx)
            pltpu.sync_copy(pos_hbm.at[pl.ds(gbase * _P, _C * _P)], pidx)
            pltpu.sync_copy(neg_hbm.at[pl.ds(gbase * _N, _C * _N)], nidx)
            _remap(cidx, _C)
            _remap(pidx, _C * _P)
            _remap(nidx, _C * _N)

            copies = [pltpu.async_copy(in_embed.at[cidx], crows, sem)]
            for o in range(0, _C * _P, 128):
                s = min(128, _C * _P - o)
                copies.append(pltpu.async_copy(
                    out_embed.at[pidx.at[pl.ds(o, s)]], prows.at[pl.ds(o, s)],
                    sem))
            for o in range(0, _C * _N, 128):
                s = min(128, _C * _N - o)
                copies.append(pltpu.async_copy(
                    out_embed.at[nidx.at[pl.ds(o, s)]], nrows.at[pl.ds(o, s)],
                    sem))
            for cp in copies:
                cp.wait()

            def dots(bb, _):
                v = [crows[bb, pl.ds(k * 16, 16)] for k in range(_DIM // 16)]
                col = jnp.full((16,), h * _C, jnp.int32) + bb
                for j in range(_NPAIR):
                    if j < _P:
                        row = bb * _P + j
                        src = prows
                    else:
                        row = bb * _N + (j - _P)
                        src = nrows
                    acc = v[0] * src[row, pl.ds(0, 16)]
                    for k in range(1, _DIM // 16):
                        acc = acc + v[k] * src[row, pl.ds(k * 16, 16)]
                    tot = plsc.cumsum(acc)
                    plsc.store_scatter(
                        lg, [jnp.full((16,), j, jnp.int32), col], tot,
                        mask=last)
                return _

            lax.fori_loop(0, _C, dots, 0)

        obase = pl.multiple_of(wid * _BPW + it * 2 * _C, 2 * _C)
        pltpu.sync_copy(lg, sl_out.at[:, pl.ds(obase, 2 * _C)])
        return _

    lax.fori_loop(0, _NITER // 2, body, 0)


def _sc_logits(center, pos_flat, neg_flat, in_embed, out_embed):
    mesh = plsc.VectorSubcoreMesh(core_axis_name="c", subcore_axis_name="s")
    return pl.kernel(
        _sc_logits_kernel,
        out_type=jax.ShapeDtypeStruct((_ROWS, _B), jnp.float32),
        mesh=mesh,
        compiler_params=pltpu.CompilerParams(
            needs_layout_passes=False, use_tc_tiling_on_sc=False),
        scratch_types=[
            pltpu.VMEM((_C,), jnp.int32),
            pltpu.VMEM((_C * _P,), jnp.int32),
            pltpu.VMEM((_C * _N,), jnp.int32),
            pltpu.VMEM((_C, _DIM), jnp.float32),
            pltpu.VMEM((_C * _P, _DIM), jnp.float32),
            pltpu.VMEM((_C * _N, _DIM), jnp.float32),
            pltpu.VMEM((_ROWS, 2 * _C), jnp.float32),
            pltpu.SemaphoreType.DMA,
        ],
    )(center, pos_flat, neg_flat, in_embed, out_embed)


def _tc_loss_kernel(sl_ref, out_ref):
    x = sl_ref[...]
    row = lax.broadcasted_iota(jnp.int32, x.shape, 0)
    x = jnp.where(row < _P, x, -x)
    ls = jnp.where(row < _NPAIR, jax.nn.log_sigmoid(x), 0.0)
    out_ref[0, 0] = -jnp.sum(ls) / _B


def _tc_loss(sl):
    out = pl.pallas_call(
        _tc_loss_kernel,
        out_shape=jax.ShapeDtypeStruct((1, 1), jnp.float32),
        out_specs=pl.BlockSpec(memory_space=pltpu.SMEM),
    )(sl)
    return out[0, 0]


@jax.jit
def kernel(center, pos, neg, in_embed, out_embed):
    in_h, out_h = _transpose_tables(in_embed.T, out_embed.T)
    in_row = in_h.reshape(2 * _H, _DIM)
    out_row = out_h.reshape(2 * _H, _DIM)
    sl = _sc_logits(center, pos.reshape(-1), neg.reshape(-1), in_row, out_row)
    return _tc_loss(sl)


# speedup vs baseline: 1.1171x; 1.1171x over previous
import functools

import jax
import jax.numpy as jnp
from jax import lax
from jax.experimental import pallas as pl
from jax.experimental.pallas import tpu as pltpu
from jax.experimental.pallas import tpu_sc as plsc

_VOCAB = 1000000
_DIM = 64
_B = 16384
_P = 5
_N = 20
_NPAIR = _P + _N
_ROWS = 32

_NC = 2
_NS = 16
_NW = _NC * _NS
_BPW = _B // _NW
_C = 64
_NITER = _BPW // _C

_TV = 8192
_TGRID = 62
_H = _TV * _TGRID


def _tr_kernel(a1_ref, a2_ref, b1_ref, b2_ref, ao_ref, bo_ref):
    eye = jnp.eye(_DIM, dtype=jnp.float32)

    def tr(x):
        return lax.dot_general(
            x, eye, (((0,), (0,)), ((), ())),
            preferred_element_type=jnp.float32)

    ao_ref[...] = jnp.concatenate([tr(a1_ref[...]), tr(a2_ref[...])], axis=1)
    bo_ref[...] = jnp.concatenate([tr(b1_ref[...]), tr(b2_ref[...])], axis=1)


def _transpose_tables(in_t, out_t):
    nblk = pl.cdiv(_VOCAB, _TV)
    lo_spec = pl.BlockSpec((_DIM, _TV), lambda i: (0, i))
    hi_spec = pl.BlockSpec(
        (_DIM, _TV), lambda i: (0, jnp.minimum(_TGRID + i, nblk - 1)))
    out_spec = pl.BlockSpec((_TV, 2 * _DIM), lambda i: (i, 0))
    return pl.pallas_call(
        _tr_kernel,
        grid=(_TGRID,),
        in_specs=[lo_spec, hi_spec, lo_spec, hi_spec],
        out_specs=[out_spec, out_spec],
        out_shape=[
            jax.ShapeDtypeStruct((_H, 2 * _DIM), jnp.float32),
            jax.ShapeDtypeStruct((_H, 2 * _DIM), jnp.float32),
        ],
    )(in_t, in_t, out_t, out_t)


def _remap(buf, n):
    def body(i, _):
        v = buf[pl.ds(i * 16, 16)]
        r = v + v
        buf[pl.ds(i * 16, 16)] = jnp.where(v < _H, r, r - (2 * _H - 1))
        return _
    lax.fori_loop(0, n // 16, body, 0)


_LANE15 = 15


def _sc_logits_kernel(center_hbm, pos_hbm, neg_hbm, in_embed, out_embed,
                      sl_out, cidx, pidx, nidx, crows, prows, nrows, lg, sem):
    wid = lax.axis_index("s") * _NC + lax.axis_index("c")

    zero16 = jnp.zeros((16,), jnp.float32)
    for r in range(_NPAIR, _ROWS):
        for c in range(2 * _C // 16):
            lg[r, pl.ds(c * 16, 16)] = zero16

    lane = lax.iota(jnp.int32, 16)
    last = lane == _LANE15

    def body(it, _):
        for h in range(2):
            t = it * 2 + h
            gbase = pl.multiple_of(wid * _BPW + t * _C, _C)

            pltpu.sync_copy(center_hbm.at[pl.ds(gbase, _C)], cidx)
            pltpu.sync_copy(pos_hbm.at[pl.ds(gbase * _P, _C * _P)], pidx)
            pltpu.sync_copy(neg_hbm.at[pl.ds(gbase * _N, _C * _N)], nidx)
            _remap(cidx, _C)
            _remap(pidx, _C * _P)
            _remap(nidx, _C * _N)

            copies = [pltpu.async_copy(in_embed.at[cidx], crows, sem)]
            for o in range(0, _C * _P, 128):
                s = min(128, _C * _P - o)
                copies.append(pltpu.async_copy(
                    out_embed.at[pidx.at[pl.ds(o, s)]], prows.at[pl.ds(o, s)],
                    sem))
            for o in range(0, _C * _N, 128):
                s = min(128, _C * _N - o)
                copies.append(pltpu.async_copy(
                    out_embed.at[nidx.at[pl.ds(o, s)]], nrows.at[pl.ds(o, s)],
                    sem))
            for cp in copies:
                cp.wait()

            def dots(bb, _):
                v = [crows[bb, pl.ds(k * 16, 16)] for k in range(_DIM // 16)]
                col = jnp.full((16,), h * _C, jnp.int32) + bb
                for j in range(_NPAIR):
                    if j < _P:
                        row = bb * _P + j
                        src = prows
                    else:
                        row = bb * _N + (j - _P)
                        src = nrows
                    acc = v[0] * src[row, pl.ds(0, 16)]
                    for k in range(1, _DIM // 16):
                        acc = acc + v[k] * src[row, pl.ds(k * 16, 16)]
                    tot = plsc.cumsum(acc)
                    plsc.store_scatter(
                        lg, [jnp.full((16,), j, jnp.int32), col], tot,
                        mask=last)
                return _

            lax.fori_loop(0, _C, dots, 0)

        obase = pl.multiple_of(wid * _BPW + it * 2 * _C, 2 * _C)
        pltpu.sync_copy(lg, sl_out.at[:, pl.ds(obase, 2 * _C)])
        return _

    lax.fori_loop(0, _NITER // 2, body, 0)


def _sc_logits(center, pos_flat, neg_flat, in_embed, out_embed):
    mesh = plsc.VectorSubcoreMesh(core_axis_name="c", subcore_axis_name="s")
    return pl.kernel(
        _sc_logits_kernel,
        out_type=jax.ShapeDtypeStruct((_ROWS, _B), jnp.float32),
        mesh=mesh,
        compiler_params=pltpu.CompilerParams(
            needs_layout_passes=False, use_tc_tiling_on_sc=False),
        scratch_types=[
            pltpu.VMEM((_C,), jnp.int32),
            pltpu.VMEM((_C * _P,), jnp.int32),
            pltpu.VMEM((_C * _N,), jnp.int32),
            pltpu.VMEM((_C, _DIM), jnp.float32),
            pltpu.VMEM((_C * _P, _DIM), jnp.float32),
            pltpu.VMEM((_C * _N, _DIM), jnp.float32),
            pltpu.VMEM((_ROWS, 2 * _C), jnp.float32),
            pltpu.SemaphoreType.DMA,
        ],
    )(center, pos_flat, neg_flat, in_embed, out_embed)


def _tc_loss_kernel(sl_ref, out_ref):
    x = sl_ref[...]
    row = lax.broadcasted_iota(jnp.int32, x.shape, 0)
    x = jnp.where(row < _P, x, -x)
    ls = jnp.where(row < _NPAIR, jax.nn.log_sigmoid(x), 0.0)
    out_ref[0, 0] = -jnp.sum(ls) / _B


def _tc_loss(sl):
    out = pl.pallas_call(
        _tc_loss_kernel,
        out_shape=jax.ShapeDtypeStruct((1, 1), jnp.float32),
        out_specs=pl.BlockSpec(memory_space=pltpu.SMEM),
    )(sl)
    return out[0, 0]


@jax.jit
def kernel(center, pos, neg, in_embed, out_embed):
    in_h, out_h = _transpose_tables(in_embed.T, out_embed.T)
    in_row = in_h.reshape(2 * _H, _DIM)
    out_row = out_h.reshape(2 * _H, _DIM)
    sl = _sc_logits(center, pos.reshape(-1), neg.reshape(-1), in_row, out_row)
    return _tc_loss(sl)


# speedup vs baseline: 1.1608x; 1.0391x over previous
import functools

import jax
import jax.numpy as jnp
from jax import lax
from jax.experimental import pallas as pl
from jax.experimental.pallas import tpu as pltpu
from jax.experimental.pallas import tpu_sc as plsc

_VOCAB = 1000000
_DIM = 64
_B = 16384
_P = 5
_N = 20
_NPAIR = _P + _N
_ROWS = 32

_NC = 2
_NS = 16
_NW = _NC * _NS
_BPW = _B // _NW
_C = 32
_NCH = _BPW // _C

_TV = 8192
_TGRID = 62
_H = _TV * _TGRID


def _tr_kernel(a1_ref, a2_ref, b1_ref, b2_ref, ao_ref, bo_ref):
    eye = jnp.eye(_DIM, dtype=jnp.float32)

    def tr(x):
        return lax.dot_general(
            x, eye, (((0,), (0,)), ((), ())),
            preferred_element_type=jnp.float32)

    ao_ref[...] = jnp.concatenate([tr(a1_ref[...]), tr(a2_ref[...])], axis=1)
    bo_ref[...] = jnp.concatenate([tr(b1_ref[...]), tr(b2_ref[...])], axis=1)


def _transpose_tables(in_t, out_t):
    nblk = pl.cdiv(_VOCAB, _TV)
    lo_spec = pl.BlockSpec((_DIM, _TV), lambda i: (0, i))
    hi_spec = pl.BlockSpec(
        (_DIM, _TV), lambda i: (0, jnp.minimum(_TGRID + i, nblk - 1)))
    out_spec = pl.BlockSpec((_TV, 2 * _DIM), lambda i: (i, 0))
    return pl.pallas_call(
        _tr_kernel,
        grid=(_TGRID,),
        compiler_params=pltpu.CompilerParams(
            vmem_limit_bytes=120 * 1024 * 1024),
        in_specs=[lo_spec, hi_spec, lo_spec, hi_spec],
        out_specs=[out_spec, out_spec],
        out_shape=[
            jax.ShapeDtypeStruct((_H, 2 * _DIM), jnp.float32),
            jax.ShapeDtypeStruct((_H, 2 * _DIM), jnp.float32),
        ],
    )(in_t, in_t, out_t, out_t)


def _remap(buf, n):
    def body(i, _):
        v = buf[pl.ds(i * 16, 16)]
        r = v + v
        buf[pl.ds(i * 16, 16)] = jnp.where(v < _H, r, r - (2 * _H - 1))
        return _
    lax.fori_loop(0, n // 16, body, 0)


_LANE15 = 15


def _sc_logits_kernel(center_hbm, pos_hbm, neg_hbm, in_embed, out_embed,
                      sl_out,
                      cidx0, pidx0, nidx0, crows0, prows0, nrows0, sem0,
                      cidx1, pidx1, nidx1, crows1, prows1, nrows1, sem1,
                      lg):
    wid = lax.axis_index("s") * _NC + lax.axis_index("c")
    slots = ((cidx0, pidx0, nidx0, crows0, prows0, nrows0, sem0),
             (cidx1, pidx1, nidx1, crows1, prows1, nrows1, sem1))

    zero16 = jnp.zeros((16,), jnp.float32)
    for r in range(_NPAIR, _ROWS):
        for c in range(4 * _C // 16):
            lg[r, pl.ds(c * 16, 16)] = zero16

    lane = lax.iota(jnp.int32, 16)
    last = lane == _LANE15

    def gather_list(slot):
        cidx, pidx, nidx, crows, prows, nrows, sem = slots[slot]
        gl = [(in_embed.at[cidx], crows, sem)]
        for o in range(0, _C * _P, 128):
            s = min(128, _C * _P - o)
            gl.append((out_embed.at[pidx.at[pl.ds(o, s)]],
                       prows.at[pl.ds(o, s)], sem))
        for o in range(0, _C * _N, 128):
            s = min(128, _C * _N - o)
            gl.append((out_embed.at[nidx.at[pl.ds(o, s)]],
                       nrows.at[pl.ds(o, s)], sem))
        return gl

    def stage(c, slot):
        cidx, pidx, nidx, crows, prows, nrows, sem = slots[slot]
        gbase = pl.multiple_of(wid * _BPW + c * _C, _C)
        pltpu.sync_copy(center_hbm.at[pl.ds(gbase, _C)], cidx)
        pltpu.sync_copy(pos_hbm.at[pl.ds(gbase * _P, _C * _P)], pidx)
        pltpu.sync_copy(neg_hbm.at[pl.ds(gbase * _N, _C * _N)], nidx)
        _remap(cidx, _C)
        _remap(pidx, _C * _P)
        _remap(nidx, _C * _N)
        for src, dst, sem_ in gather_list(slot):
            pltpu.async_copy(src, dst, sem_)

    def drain(slot):
        for src, dst, sem_ in gather_list(slot):
            pltpu.make_async_copy(src, dst, sem_).wait()

    def compute(slot, col0):
        _, _, _, crows, prows, nrows, _ = slots[slot]

        def dots(bb, _):
            v = [crows[bb, pl.ds(k * 16, 16)] for k in range(_DIM // 16)]
            col = jnp.full((16,), col0, jnp.int32) + bb
            for j in range(_NPAIR):
                if j < _P:
                    row = bb * _P + j
                    src = prows
                else:
                    row = bb * _N + (j - _P)
                    src = nrows
                acc = v[0] * src[row, pl.ds(0, 16)]
                for k in range(1, _DIM // 16):
                    acc = acc + v[k] * src[row, pl.ds(k * 16, 16)]
                tot = plsc.cumsum(acc)
                plsc.store_scatter(
                    lg, [jnp.full((16,), j, jnp.int32), col], tot, mask=last)
            return _

        lax.fori_loop(0, _C, dots, 0)

    stage(0, 0)

    def body(it, _):
        for h in range(4):
            c = it * 4 + h
            slot = h % 2

            @pl.when(c < _NCH - 1)
            def _prefetch():
                stage(c + 1, 1 - slot)

            drain(slot)
            compute(slot, h * _C)

        obase = pl.multiple_of(wid * _BPW + it * 4 * _C, 4 * _C)
        pltpu.sync_copy(lg, sl_out.at[:, pl.ds(obase, 4 * _C)])
        return _

    lax.fori_loop(0, _NCH // 4, body, 0)


def _sc_logits(center, pos_flat, neg_flat, in_embed, out_embed):
    mesh = plsc.VectorSubcoreMesh(core_axis_name="c", subcore_axis_name="s")
    return pl.kernel(
        _sc_logits_kernel,
        out_type=jax.ShapeDtypeStruct((_ROWS, _B), jnp.float32),
        mesh=mesh,
        compiler_params=pltpu.CompilerParams(
            needs_layout_passes=False, use_tc_tiling_on_sc=False),
        scratch_types=(
            [
                pltpu.VMEM((_C,), jnp.int32),
                pltpu.VMEM((_C * _P,), jnp.int32),
                pltpu.VMEM((_C * _N,), jnp.int32),
                pltpu.VMEM((_C, _DIM), jnp.float32),
                pltpu.VMEM((_C * _P, _DIM), jnp.float32),
                pltpu.VMEM((_C * _N, _DIM), jnp.float32),
                pltpu.SemaphoreType.DMA,
            ] * 2
            + [pltpu.VMEM((_ROWS, 4 * _C), jnp.float32)]
        ),
    )(center, pos_flat, neg_flat, in_embed, out_embed)


def _tc_loss_kernel(sl_ref, out_ref):
    x = sl_ref[...]
    row = lax.broadcasted_iota(jnp.int32, x.shape, 0)
    x = jnp.where(row < _P, x, -x)
    ls = jnp.where(row < _NPAIR, jax.nn.log_sigmoid(x), 0.0)
    out_ref[0, 0] = -jnp.sum(ls) / _B


def _tc_loss(sl):
    out = pl.pallas_call(
        _tc_loss_kernel,
        out_shape=jax.ShapeDtypeStruct((1, 1), jnp.float32),
        out_specs=pl.BlockSpec(memory_space=pltpu.SMEM),
    )(sl)
    return out[0, 0]


@jax.jit
def kernel(center, pos, neg, in_embed, out_embed):
    in_h, out_h = _transpose_tables(in_embed.T, out_embed.T)
    in_row = in_h.reshape(2 * _H, _DIM)
    out_row = out_h.reshape(2 * _H, _DIM)
    sl = _sc_logits(center, pos.reshape(-1), neg.reshape(-1), in_row, out_row)
    return _tc_loss(sl)


# speedup vs baseline: 1.1890x; 1.0243x over previous
import functools

import jax
import jax.numpy as jnp
from jax import lax
from jax.experimental import pallas as pl
from jax.experimental.pallas import tpu as pltpu
from jax.experimental.pallas import tpu_sc as plsc

_VOCAB = 1000000
_DIM = 64
_B = 16384
_P = 5
_N = 20
_NPAIR = _P + _N
_ROWS = 32

_NC = 2
_NS = 16
_NW = _NC * _NS
_BPW = _B // _NW
_C = 32
_NCH = _BPW // _C

_TV = 8192
_TGRID = 62
_H = _TV * _TGRID


def _tr_kernel(a1_ref, a2_ref, b1_ref, b2_ref, ao_ref, bo_ref):
    eye = jnp.eye(_DIM, dtype=jnp.float32)

    def tr(x):
        return lax.dot_general(
            x, eye, (((0,), (0,)), ((), ())),
            preferred_element_type=jnp.float32)

    ao_ref[...] = jnp.concatenate([tr(a1_ref[...]), tr(a2_ref[...])], axis=1)
    bo_ref[...] = jnp.concatenate([tr(b1_ref[...]), tr(b2_ref[...])], axis=1)


def _transpose_tables(in_t, out_t):
    nblk = pl.cdiv(_VOCAB, _TV)
    lo_spec = pl.BlockSpec((_DIM, _TV), lambda i: (0, i))
    hi_spec = pl.BlockSpec(
        (_DIM, _TV), lambda i: (0, jnp.minimum(_TGRID + i, nblk - 1)))
    out_spec = pl.BlockSpec((_TV, 2 * _DIM), lambda i: (i, 0))
    return pl.pallas_call(
        _tr_kernel,
        grid=(_TGRID,),
        compiler_params=pltpu.CompilerParams(
            vmem_limit_bytes=120 * 1024 * 1024),
        in_specs=[lo_spec, hi_spec, lo_spec, hi_spec],
        out_specs=[out_spec, out_spec],
        out_shape=[
            jax.ShapeDtypeStruct((_H, 2 * _DIM), jnp.float32),
            jax.ShapeDtypeStruct((_H, 2 * _DIM), jnp.float32),
        ],
    )(in_t, in_t, out_t, out_t)


def _remap(buf, n):
    def body(i, _):
        v = buf[pl.ds(i * 16, 16)]
        r = v + v
        buf[pl.ds(i * 16, 16)] = jnp.where(v < _H, r, r - (2 * _H - 1))
        return _
    lax.fori_loop(0, n // 16, body, 0)


_LANE15 = 15


def _sc_logits_kernel(center_hbm, pos_hbm, neg_hbm, in_embed, out_embed,
                      sl_out, cidx, pidx, nidx,
                      crows0, prows0, nrows0, sem0,
                      crows1, prows1, nrows1, sem1,
                      lg):
    wid = lax.axis_index("s") * _NC + lax.axis_index("c")
    slots = ((crows0, prows0, nrows0, sem0),
             (crows1, prows1, nrows1, sem1))

    pltpu.sync_copy(center_hbm.at[pl.ds(wid * _BPW, _BPW)], cidx)
    pltpu.sync_copy(pos_hbm.at[pl.ds(wid * _BPW * _P, _BPW * _P)], pidx)
    pltpu.sync_copy(neg_hbm.at[pl.ds(wid * _BPW * _N, _BPW * _N)], nidx)
    _remap(cidx, _BPW)
    _remap(pidx, _BPW * _P)
    _remap(nidx, _BPW * _N)

    zero16 = jnp.zeros((16,), jnp.float32)
    for r in range(_NPAIR, _ROWS):
        for c in range(4 * _C // 16):
            lg[r, pl.ds(c * 16, 16)] = zero16

    lane = lax.iota(jnp.int32, 16)
    last = lane == _LANE15

    def gather_list(c, slot):
        crows, prows, nrows, sem = slots[slot]
        gl = [(in_embed.at[cidx.at[pl.ds(c * _C, _C)]], crows, sem)]
        for o in range(0, _C * _P, 128):
            s = min(128, _C * _P - o)
            gl.append((out_embed.at[pidx.at[pl.ds(c * _C * _P + o, s)]],
                       prows.at[pl.ds(o, s)], sem))
        for o in range(0, _C * _N, 128):
            s = min(128, _C * _N - o)
            gl.append((out_embed.at[nidx.at[pl.ds(c * _C * _N + o, s)]],
                       nrows.at[pl.ds(o, s)], sem))
        return gl

    def stage(c, slot):
        for src, dst, sem_ in gather_list(c, slot):
            pltpu.async_copy(src, dst, sem_)

    def drain(c, slot):
        for src, dst, sem_ in gather_list(c, slot):
            pltpu.make_async_copy(src, dst, sem_).wait()

    def compute(slot, col0):
        crows, prows, nrows, _ = slots[slot]

        def dots(bb, _):
            v = [crows[bb, pl.ds(k * 16, 16)] for k in range(_DIM // 16)]
            col = jnp.full((16,), col0, jnp.int32) + bb
            for j in range(_NPAIR):
                if j < _P:
                    row = bb * _P + j
                    src = prows
                else:
                    row = bb * _N + (j - _P)
                    src = nrows
                acc = v[0] * src[row, pl.ds(0, 16)]
                for k in range(1, _DIM // 16):
                    acc = acc + v[k] * src[row, pl.ds(k * 16, 16)]
                tot = plsc.cumsum(acc)
                plsc.store_scatter(
                    lg, [jnp.full((16,), j, jnp.int32), col], tot, mask=last)
            return _

        lax.fori_loop(0, _C, dots, 0)

    stage(0, 0)

    def body(it, _):
        for h in range(4):
            c = it * 4 + h
            slot = h % 2

            @pl.when(c < _NCH - 1)
            def _prefetch():
                stage(c + 1, 1 - slot)

            drain(c, slot)
            compute(slot, h * _C)

        obase = pl.multiple_of(wid * _BPW + it * 4 * _C, 4 * _C)
        pltpu.sync_copy(lg, sl_out.at[:, pl.ds(obase, 4 * _C)])
        return _

    lax.fori_loop(0, _NCH // 4, body, 0)


def _sc_logits(center, pos_flat, neg_flat, in_embed, out_embed):
    mesh = plsc.VectorSubcoreMesh(core_axis_name="c", subcore_axis_name="s")
    return pl.kernel(
        _sc_logits_kernel,
        out_type=jax.ShapeDtypeStruct((_ROWS, _B), jnp.float32),
        mesh=mesh,
        compiler_params=pltpu.CompilerParams(
            needs_layout_passes=False, use_tc_tiling_on_sc=False),
        scratch_types=(
            [
                pltpu.VMEM((_BPW,), jnp.int32),
                pltpu.VMEM((_BPW * _P,), jnp.int32),
                pltpu.VMEM((_BPW * _N,), jnp.int32),
            ]
            + [
                pltpu.VMEM((_C, _DIM), jnp.float32),
                pltpu.VMEM((_C * _P, _DIM), jnp.float32),
                pltpu.VMEM((_C * _N, _DIM), jnp.float32),
                pltpu.SemaphoreType.DMA,
            ] * 2
            + [pltpu.VMEM((_ROWS, 4 * _C), jnp.float32)]
        ),
    )(center, pos_flat, neg_flat, in_embed, out_embed)


def _tc_loss_kernel(sl_ref, out_ref):
    x = sl_ref[...]
    row = lax.broadcasted_iota(jnp.int32, x.shape, 0)
    x = jnp.where(row < _P, x, -x)
    ls = jnp.where(row < _NPAIR, jax.nn.log_sigmoid(x), 0.0)
    out_ref[0, 0] = -jnp.sum(ls) / _B


def _tc_loss(sl):
    out = pl.pallas_call(
        _tc_loss_kernel,
        out_shape=jax.ShapeDtypeStruct((1, 1), jnp.float32),
        out_specs=pl.BlockSpec(memory_space=pltpu.SMEM),
    )(sl)
    return out[0, 0]


@jax.jit
def kernel(center, pos, neg, in_embed, out_embed):
    in_h, out_h = _transpose_tables(in_embed.T, out_embed.T)
    in_row = in_h.reshape(2 * _H, _DIM)
    out_row = out_h.reshape(2 * _H, _DIM)
    sl = _sc_logits(center, pos.reshape(-1), neg.reshape(-1), in_row, out_row)
    return _tc_loss(sl)


# speedup vs baseline: 1.1996x; 1.0089x over previous
import functools

import jax
import jax.numpy as jnp
from jax import lax
from jax.experimental import pallas as pl
from jax.experimental.pallas import tpu as pltpu
from jax.experimental.pallas import tpu_sc as plsc

_VOCAB = 1000000
_DIM = 64
_B = 16384
_P = 5
_N = 20
_NPAIR = _P + _N
_ROWS = 32

_NC = 2
_NS = 16
_NW = _NC * _NS
_BPW = _B // _NW
_C = 32
_NCH = _BPW // _C

_TV = 10240
_TGRID = 49
_H = _TV * _TGRID


def _tr_kernel(a1_ref, a2_ref, b1_ref, b2_ref, ao_ref, bo_ref):
    def tr(x):
        return jnp.swapaxes(x, 0, 1)

    ao_ref[...] = jnp.concatenate([tr(a1_ref[...]), tr(a2_ref[...])], axis=1)
    bo_ref[...] = jnp.concatenate([tr(b1_ref[...]), tr(b2_ref[...])], axis=1)


def _transpose_tables(in_t, out_t):
    nblk = pl.cdiv(_VOCAB, _TV)
    lo_spec = pl.BlockSpec((_DIM, _TV), lambda i: (0, i))
    hi_spec = pl.BlockSpec(
        (_DIM, _TV), lambda i: (0, jnp.minimum(_TGRID + i, nblk - 1)))
    out_spec = pl.BlockSpec((_TV, 2 * _DIM), lambda i: (i, 0))
    return pl.pallas_call(
        _tr_kernel,
        grid=(_TGRID,),
        compiler_params=pltpu.CompilerParams(
            vmem_limit_bytes=120 * 1024 * 1024),
        in_specs=[lo_spec, hi_spec, lo_spec, hi_spec],
        out_specs=[out_spec, out_spec],
        out_shape=[
            jax.ShapeDtypeStruct((_H, 2 * _DIM), jnp.float32),
            jax.ShapeDtypeStruct((_H, 2 * _DIM), jnp.float32),
        ],
    )(in_t, in_t, out_t, out_t)


def _remap(buf, n):
    def body(i, _):
        v = buf[pl.ds(i * 16, 16)]
        r = v + v
        buf[pl.ds(i * 16, 16)] = jnp.where(v < _H, r, r - (2 * _H - 1))
        return _
    lax.fori_loop(0, n // 16, body, 0)


_LANE15 = 15


def _sc_logits_kernel(center_hbm, pos_hbm, neg_hbm, in_embed, out_embed,
                      sl_out, cidx, pidx, nidx,
                      crows0, prows0, nrows0, sem0,
                      crows1, prows1, nrows1, sem1,
                      lg):
    wid = lax.axis_index("s") * _NC + lax.axis_index("c")
    slots = ((crows0, prows0, nrows0, sem0),
             (crows1, prows1, nrows1, sem1))

    pltpu.sync_copy(center_hbm.at[pl.ds(wid * _BPW, _BPW)], cidx)
    pltpu.sync_copy(pos_hbm.at[pl.ds(wid * _BPW * _P, _BPW * _P)], pidx)
    pltpu.sync_copy(neg_hbm.at[pl.ds(wid * _BPW * _N, _BPW * _N)], nidx)
    _remap(cidx, _BPW)
    _remap(pidx, _BPW * _P)
    _remap(nidx, _BPW * _N)

    zero16 = jnp.zeros((16,), jnp.float32)
    for r in range(_NPAIR, _ROWS):
        for c in range(4 * _C // 16):
            lg[r, pl.ds(c * 16, 16)] = zero16

    lane = lax.iota(jnp.int32, 16)
    last = lane == _LANE15

    def gather_list(c, slot):
        crows, prows, nrows, sem = slots[slot]
        gl = [(in_embed.at[cidx.at[pl.ds(c * _C, _C)]], crows, sem)]
        for o in range(0, _C * _P, 128):
            s = min(128, _C * _P - o)
            gl.append((out_embed.at[pidx.at[pl.ds(c * _C * _P + o, s)]],
                       prows.at[pl.ds(o, s)], sem))
        for o in range(0, _C * _N, 128):
            s = min(128, _C * _N - o)
            gl.append((out_embed.at[nidx.at[pl.ds(c * _C * _N + o, s)]],
                       nrows.at[pl.ds(o, s)], sem))
        return gl

    def stage(c, slot):
        for src, dst, sem_ in gather_list(c, slot):
            pltpu.async_copy(src, dst, sem_)

    def drain(c, slot):
        for src, dst, sem_ in gather_list(c, slot):
            pltpu.make_async_copy(src, dst, sem_).wait()

    def compute(slot, col0):
        crows, prows, nrows, _ = slots[slot]

        def dots(bb, _):
            v = [crows[bb, pl.ds(k * 16, 16)] for k in range(_DIM // 16)]
            col = jnp.full((16,), col0, jnp.int32) + bb
            for j in range(_NPAIR):
                if j < _P:
                    row = bb * _P + j
                    src = prows
                else:
                    row = bb * _N + (j - _P)
                    src = nrows
                acc = v[0] * src[row, pl.ds(0, 16)]
                for k in range(1, _DIM // 16):
                    acc = acc + v[k] * src[row, pl.ds(k * 16, 16)]
                tot = plsc.cumsum(acc)
                plsc.store_scatter(
                    lg, [jnp.full((16,), j, jnp.int32), col], tot, mask=last)
            return _

        lax.fori_loop(0, _C, dots, 0)

    stage(0, 0)

    def body(it, _):
        for h in range(4):
            c = it * 4 + h
            slot = h % 2

            @pl.when(c < _NCH - 1)
            def _prefetch():
                stage(c + 1, 1 - slot)

            drain(c, slot)
            compute(slot, h * _C)

        obase = pl.multiple_of(wid * _BPW + it * 4 * _C, 4 * _C)
        pltpu.sync_copy(lg, sl_out.at[:, pl.ds(obase, 4 * _C)])
        return _

    lax.fori_loop(0, _NCH // 4, body, 0)


def _sc_logits(center, pos_flat, neg_flat, in_embed, out_embed):
    mesh = plsc.VectorSubcoreMesh(core_axis_name="c", subcore_axis_name="s")
    return pl.kernel(
        _sc_logits_kernel,
        out_type=jax.ShapeDtypeStruct((_ROWS, _B), jnp.float32),
        mesh=mesh,
        compiler_params=pltpu.CompilerParams(
            needs_layout_passes=False, use_tc_tiling_on_sc=False),
        scratch_types=(
            [
                pltpu.VMEM((_BPW,), jnp.int32),
                pltpu.VMEM((_BPW * _P,), jnp.int32),
                pltpu.VMEM((_BPW * _N,), jnp.int32),
            ]
            + [
                pltpu.VMEM((_C, _DIM), jnp.float32),
                pltpu.VMEM((_C * _P, _DIM), jnp.float32),
                pltpu.VMEM((_C * _N, _DIM), jnp.float32),
                pltpu.SemaphoreType.DMA,
            ] * 2
            + [pltpu.VMEM((_ROWS, 4 * _C), jnp.float32)]
        ),
    )(center, pos_flat, neg_flat, in_embed, out_embed)


def _tc_loss_kernel(sl_ref, out_ref):
    x = sl_ref[...]
    row = lax.broadcasted_iota(jnp.int32, x.shape, 0)
    x = jnp.where(row < _P, x, -x)
    ls = jnp.where(row < _NPAIR, jax.nn.log_sigmoid(x), 0.0)
    out_ref[0, 0] = -jnp.sum(ls) / _B


def _tc_loss(sl):
    out = pl.pallas_call(
        _tc_loss_kernel,
        out_shape=jax.ShapeDtypeStruct((1, 1), jnp.float32),
        out_specs=pl.BlockSpec(memory_space=pltpu.SMEM),
    )(sl)
    return out[0, 0]


@jax.jit
def kernel(center, pos, neg, in_embed, out_embed):
    in_h, out_h = _transpose_tables(in_embed.T, out_embed.T)
    in_row = in_h.reshape(2 * _H, _DIM)
    out_row = out_h.reshape(2 * _H, _DIM)
    sl = _sc_logits(center, pos.reshape(-1), neg.reshape(-1), in_row, out_row)
    return _tc_loss(sl)


# speedup vs baseline: 1.2001x; 1.0004x over previous
import jax
import jax.numpy as jnp
from jax import lax
from jax.experimental import pallas as pl
from jax.experimental.pallas import tpu as pltpu
from jax.experimental.pallas import tpu_sc as plsc

_VOCAB = 1000000
_DIM = 64
_B = 16384
_P = 5
_N = 20
_NPAIR = _P + _N
_ROWS = 32

_NC = 2
_NS = 16
_NW = _NC * _NS
_BPW = _B // _NW
_C = 32
_NCH = _BPW // _C

_TV = 10240
_TGRID = 49
_H = _TV * _TGRID


def _tr_kernel(a1_ref, a2_ref, b1_ref, b2_ref, ao_ref, bo_ref):
    def tr(x):
        return jnp.swapaxes(x, 0, 1)

    ao_ref[...] = jnp.concatenate([tr(a1_ref[...]), tr(a2_ref[...])], axis=1)
    bo_ref[...] = jnp.concatenate([tr(b1_ref[...]), tr(b2_ref[...])], axis=1)


def _transpose_tables(in_t, out_t):
    nblk = pl.cdiv(_VOCAB, _TV)
    lo_spec = pl.BlockSpec((_DIM, _TV), lambda i: (0, i))
    hi_spec = pl.BlockSpec(
        (_DIM, _TV), lambda i: (0, jnp.minimum(_TGRID + i, nblk - 1)))
    out_spec = pl.BlockSpec((_TV, 2 * _DIM), lambda i: (i, 0))
    return pl.pallas_call(
        _tr_kernel,
        grid=(_TGRID,),
        compiler_params=pltpu.CompilerParams(
            vmem_limit_bytes=120 * 1024 * 1024),
        in_specs=[lo_spec, hi_spec, lo_spec, hi_spec],
        out_specs=[out_spec, out_spec],
        out_shape=[
            jax.ShapeDtypeStruct((_H, 2 * _DIM), jnp.float32),
            jax.ShapeDtypeStruct((_H, 2 * _DIM), jnp.float32),
        ],
    )(in_t, in_t, out_t, out_t)


def _remap(buf, n):
    def body(i, _):
        v = buf[pl.ds(i * 16, 16)]
        r = v + v
        buf[pl.ds(i * 16, 16)] = jnp.where(v < _H, r, r - (2 * _H - 1))
        return _
    lax.fori_loop(0, n // 16, body, 0)


_LANE15 = 15


def _sc_logits_kernel(center_hbm, pos_hbm, neg_hbm, in_embed, out_embed,
                      sl_out, cidx, pidx, nidx,
                      crows0, prows0, nrows0, sem0,
                      crows1, prows1, nrows1, sem1,
                      lg):
    wid = lax.axis_index("s") * _NC + lax.axis_index("c")
    slots = ((crows0, prows0, nrows0, sem0),
             (crows1, prows1, nrows1, sem1))

    pltpu.sync_copy(center_hbm.at[pl.ds(wid * _BPW, _BPW)], cidx)
    pltpu.sync_copy(pos_hbm.at[pl.ds(wid * _BPW * _P, _BPW * _P)], pidx)
    pltpu.sync_copy(neg_hbm.at[pl.ds(wid * _BPW * _N, _BPW * _N)], nidx)
    _remap(cidx, _BPW)
    _remap(pidx, _BPW * _P)
    _remap(nidx, _BPW * _N)

    zero16 = jnp.zeros((16,), jnp.float32)
    for r in range(_NPAIR, _ROWS):
        for c in range(4 * _C // 16):
            lg[r, pl.ds(c * 16, 16)] = zero16

    lane = lax.iota(jnp.int32, 16)
    last = lane == _LANE15

    def gather_list(c, slot):
        crows, prows, nrows, sem = slots[slot]
        gl = [(in_embed.at[cidx.at[pl.ds(c * _C, _C)]], crows, sem)]
        for o in range(0, _C * _P, 128):
            s = min(128, _C * _P - o)
            gl.append((out_embed.at[pidx.at[pl.ds(c * _C * _P + o, s)]],
                       prows.at[pl.ds(o, s)], sem))
        for o in range(0, _C * _N, 128):
            s = min(128, _C * _N - o)
            gl.append((out_embed.at[nidx.at[pl.ds(c * _C * _N + o, s)]],
                       nrows.at[pl.ds(o, s)], sem))
        return gl

    def stage(c, slot):
        for src, dst, sem_ in gather_list(c, slot):
            pltpu.async_copy(src, dst, sem_)

    def drain(c, slot):
        for src, dst, sem_ in gather_list(c, slot):
            pltpu.make_async_copy(src, dst, sem_).wait()

    def compute(slot, col0):
        crows, prows, nrows, _ = slots[slot]

        def dots(bb, _):
            v = [crows[bb, pl.ds(k * 16, 16)] for k in range(_DIM // 16)]
            col = jnp.full((16,), col0, jnp.int32) + bb
            for j in range(_NPAIR):
                if j < _P:
                    row = bb * _P + j
                    src = prows
                else:
                    row = bb * _N + (j - _P)
                    src = nrows
                acc = v[0] * src[row, pl.ds(0, 16)]
                for k in range(1, _DIM // 16):
                    acc = acc + v[k] * src[row, pl.ds(k * 16, 16)]
                tot = plsc.cumsum(acc)
                plsc.store_scatter(
                    lg, [jnp.full((16,), j, jnp.int32), col], tot, mask=last)
            return _

        lax.fori_loop(0, _C, dots, 0)

    stage(0, 0)

    def body(it, _):
        for h in range(4):
            c = it * 4 + h
            slot = h % 2

            @pl.when(c < _NCH - 1)
            def _prefetch():
                stage(c + 1, 1 - slot)

            drain(c, slot)
            compute(slot, h * _C)

        obase = pl.multiple_of(wid * _BPW + it * 4 * _C, 4 * _C)
        pltpu.sync_copy(lg, sl_out.at[:, pl.ds(obase, 4 * _C)])
        return _

    lax.fori_loop(0, _NCH // 4, body, 0)


def _sc_logits(center, pos_flat, neg_flat, in_embed, out_embed):
    mesh = plsc.VectorSubcoreMesh(core_axis_name="c", subcore_axis_name="s")
    return pl.kernel(
        _sc_logits_kernel,
        out_type=jax.ShapeDtypeStruct((_ROWS, _B), jnp.float32),
        mesh=mesh,
        compiler_params=pltpu.CompilerParams(
            needs_layout_passes=False, use_tc_tiling_on_sc=False),
        scratch_types=(
            [
                pltpu.VMEM((_BPW,), jnp.int32),
                pltpu.VMEM((_BPW * _P,), jnp.int32),
                pltpu.VMEM((_BPW * _N,), jnp.int32),
            ]
            + [
                pltpu.VMEM((_C, _DIM), jnp.float32),
                pltpu.VMEM((_C * _P, _DIM), jnp.float32),
                pltpu.VMEM((_C * _N, _DIM), jnp.float32),
                pltpu.SemaphoreType.DMA,
            ] * 2
            + [pltpu.VMEM((_ROWS, 4 * _C), jnp.float32)]
        ),
    )(center, pos_flat, neg_flat, in_embed, out_embed)


def _tc_loss_kernel(sl_ref, out_ref):
    x = sl_ref[...]
    row = lax.broadcasted_iota(jnp.int32, x.shape, 0)
    x = jnp.where(row < _P, x, -x)
    ls = jnp.where(row < _NPAIR, jax.nn.log_sigmoid(x), 0.0)
    out_ref[0, 0] = -jnp.sum(ls) / _B


def _tc_loss(sl):
    out = pl.pallas_call(
        _tc_loss_kernel,
        out_shape=jax.ShapeDtypeStruct((1, 1), jnp.float32),
        out_specs=pl.BlockSpec(memory_space=pltpu.SMEM),
    )(sl)
    return out[0, 0]


@jax.jit
def kernel(center, pos, neg, in_embed, out_embed):
    in_h, out_h = _transpose_tables(in_embed.T, out_embed.T)
    in_row = in_h.reshape(2 * _H, _DIM)
    out_row = out_h.reshape(2 * _H, _DIM)
    sl = _sc_logits(center, pos.reshape(-1), neg.reshape(-1), in_row, out_row)
    return _tc_loss(sl)
